# baseline (speedup 1.0000x reference)
import functools

import jax
import jax.numpy as jnp
from jax.experimental import pallas as pl
from jax.experimental.pallas import tpu as pltpu

N_INPUT = 20000
N_LABELS = 64
BATCH = 4096
EPS = 1e-8
B_TILE = 64

_LG_N = (
    655.1778003977308,
    651.7861284548891,
    243.1516405664637,
    40.31491809436625,
    2.5066282746310007,
)
_LG_D = (6.0, 11.0, 6.0, 1.0)


def _lgamma_pos(a):
    n = jnp.float32(_LG_N[-1])
    for c in _LG_N[-2::-1]:
        n = n * a + jnp.float32(c)
    d = jnp.float32(_LG_D[-1])
    for c in _LG_D[-2::-1]:
        d = d * a + jnp.float32(c)
    d = d * a
    t = a + 4.0
    return (a - 0.5) * jnp.log(t) - t + jnp.log(n / d)


_SP3_P = (10.449703348243359, 10.238049794415314, 2.5066282746310007)
_SP3_PS = (23.19438141728967, 15.251306343677316, 2.5066282746310007)


def _prep_kernel(pxr_ref, wt_ref, tab_ref, th_ref, c_ref):
    sp = jax.nn.softplus(wt_ref[...])
    hi = sp.astype(jnp.bfloat16)
    r1 = sp - hi.astype(jnp.float32)
    md = r1.astype(jnp.bfloat16)
    lo = (r1 - md.astype(jnp.float32)).astype(jnp.bfloat16)
    tab_ref[0:N_LABELS, :] = hi
    tab_ref[N_LABELS : 2 * N_LABELS, :] = md
    tab_ref[2 * N_LABELS :, :] = lo
    theta = jnp.exp(pxr_ref[...])
    th_ref[...] = theta
    c_ref[...] = jnp.sum(
        theta * jnp.log(theta + EPS) - _lgamma_pos(theta) - theta + 1.0
    )[None, None]


def _loss_kernel(x_ref, y_ref, tab_ref, th_ref, c_ref, out_ref):
    xb = x_ref[...]
    yb = y_ref[0, 0, :]
    labels = jax.lax.broadcasted_iota(jnp.int32, (B_TILE, 3 * N_LABELS), 1)
    onehot = (yb[:, None] == labels % N_LABELS).astype(jnp.bfloat16)
    px_scale = jax.lax.dot_general(
        onehot,
        tab_ref[...],
        dimension_numbers=(((1,), (0,)), ((), ())),
        preferred_element_type=jnp.float32,
    )

    th = th_ref[...]
    te = th + EPS
    th2 = th + 2.0

    lib = jnp.sum(xb, axis=1, keepdims=True)
    mu = lib * px_scale
    a1 = xb + th
    t1 = xb + th2
    t2 = xb + 3.0
    p1 = (_SP3_P[2] * a1 + _SP3_P[1]) * a1 + _SP3_P[0]
    q1 = a1 * (a1 + 1.0)
    p2 = (_SP3_PS[2] * xb + _SP3_PS[1]) * xb + _SP3_PS[0]
    v = xb + 1.5
    q2 = v * v - 0.25
    tm = te + mu
    w = q1 * p2
    z1 = tm * t2
    r = 1.0 / (z1 * w)
    u = t2 * w
    lga = jnp.log(t1 * u * r)
    lgb = jnp.log((mu + EPS) * (tm * w) * r)
    lgc = jnp.log((p1 * q2) * z1 * r * jax.lax.rsqrt(t1 * t2))
    contrib = a1 * lga + xb * lgb + lgc
    out_ref[0, 0, :] = -(jnp.sum(contrib, axis=1) + c_ref[0, 0])


@functools.partial(jax.jit, static_argnames=("interpret",))
def _run(x, y, px_r, W, interpret=False):
    nb = BATCH // B_TILE
    y2 = y.reshape(nb, 1, B_TILE)
    pxr2 = px_r.reshape(1, N_INPUT)
    wt = W.T

    tab, th, c = pl.pallas_call(
        _prep_kernel,
        out_shape=[
            jax.ShapeDtypeStruct((3 * N_LABELS, N_INPUT), jnp.bfloat16),
            jax.ShapeDtypeStruct((1, N_INPUT), jnp.float32),
            jax.ShapeDtypeStruct((1, 1), jnp.float32),
        ],
        interpret=interpret,
    )(pxr2, wt)

    loss = pl.pallas_call(
        _loss_kernel,
        grid=(nb,),
        in_specs=[
            pl.BlockSpec((B_TILE, N_INPUT), lambda i: (i, 0)),
            pl.BlockSpec((1, 1, B_TILE), lambda i: (i, 0, 0)),
            pl.BlockSpec((3 * N_LABELS, N_INPUT), lambda i: (0, 0)),
            pl.BlockSpec((1, N_INPUT), lambda i: (0, 0)),
            pl.BlockSpec((1, 1), lambda i: (0, 0)),
        ],
        out_specs=pl.BlockSpec((1, 1, B_TILE), lambda i: (i, 0, 0)),
        out_shape=jax.ShapeDtypeStruct((nb, 1, B_TILE), jnp.float32),
        compiler_params=pltpu.CompilerParams(
            dimension_semantics=("parallel",),
        ),
        interpret=interpret,
    )(x, y2, tab, th, c)
    return loss.reshape(BATCH)


def kernel(x, y, ind_x, px_r, W):
    loss = _run(x, y, px_r, W)
    zero = jnp.asarray(0.0, dtype=jnp.float32)
    return (loss, zero, zero)


# speedup vs baseline: 1.0096x; 1.0096x over previous
import functools

import jax
import jax.numpy as jnp
from jax.experimental import pallas as pl
from jax.experimental.pallas import tpu as pltpu

N_INPUT = 20000
N_LABELS = 64
BATCH = 4096
EPS = 1e-8
B_TILE = 64

_LG_N = (
    655.1778003977308,
    651.7861284548891,
    243.1516405664637,
    40.31491809436625,
    2.5066282746310007,
)
_LG_D = (6.0, 11.0, 6.0, 1.0)


def _lgamma_pos(a):
    n = jnp.float32(_LG_N[-1])
    for c in _LG_N[-2::-1]:
        n = n * a + jnp.float32(c)
    d = jnp.float32(_LG_D[-1])
    for c in _LG_D[-2::-1]:
        d = d * a + jnp.float32(c)
    d = d * a
    t = a + 4.0
    return (a - 0.5) * jnp.log(t) - t + jnp.log(n / d)


_SP3_P = (10.449703348243359, 10.238049794415314, 2.5066282746310007)
_SP3_PS = (23.19438141728967, 15.251306343677316, 2.5066282746310007)


def _loss_kernel(
    x_hbm, y_ref, pxr_ref, wt_ref, out_ref, xbuf, tab_ref, th_ref, c_ref, sem
):
    i = pl.program_id(0)
    nb = pl.num_programs(0)

    def _copy(block, slot):
        return pltpu.make_async_copy(
            x_hbm.at[pl.ds(block * B_TILE, B_TILE), :],
            xbuf.at[slot],
            sem.at[slot],
        )

    @pl.when(i == 0)
    def _start_first():
        _copy(0, 0).start()

    @pl.when(i + 1 < nb)
    def _start_next():
        _copy(i + 1, (i + 1) % 2).start()

    @pl.when(i == 0)
    def _init():
        sp = jax.nn.softplus(wt_ref[...])
        hi = sp.astype(jnp.bfloat16)
        r1 = sp - hi.astype(jnp.float32)
        md = r1.astype(jnp.bfloat16)
        lo = (r1 - md.astype(jnp.float32)).astype(jnp.bfloat16)
        tab_ref[0:N_LABELS, :] = hi
        tab_ref[N_LABELS : 2 * N_LABELS, :] = md
        tab_ref[2 * N_LABELS :, :] = lo
        theta = jnp.exp(pxr_ref[...])
        th_ref[...] = theta
        c_ref[0, 0] = jnp.sum(
            theta * jnp.log(theta + EPS) - _lgamma_pos(theta) - theta + 1.0
        )

    _copy(i, i % 2).wait()
    xb = xbuf[i % 2]

    yb = y_ref[0, 0, :]
    labels = jax.lax.broadcasted_iota(jnp.int32, (B_TILE, 3 * N_LABELS), 1)
    onehot = (yb[:, None] == labels % N_LABELS).astype(jnp.bfloat16)
    px_scale = jax.lax.dot_general(
        onehot,
        tab_ref[...],
        dimension_numbers=(((1,), (0,)), ((), ())),
        preferred_element_type=jnp.float32,
    )

    th = th_ref[...]
    te = th + EPS
    th2 = th + 2.0

    lib = jnp.sum(xb, axis=1, keepdims=True)
    mu = lib * px_scale
    a1 = xb + th
    t1 = xb + th2
    t2 = xb + 3.0
    p1 = (_SP3_P[2] * a1 + _SP3_P[1]) * a1 + _SP3_P[0]
    q1 = a1 * (a1 + 1.0)
    p2 = (_SP3_PS[2] * xb + _SP3_PS[1]) * xb + _SP3_PS[0]
    v = xb + 1.5
    q2 = v * v - 0.25
    tm = te + mu
    w = q1 * p2
    z1 = tm * t2
    r = 1.0 / (z1 * w)
    u = t2 * w
    lga = jnp.log(t1 * u * r)
    lgb = jnp.log((mu + EPS) * (tm * w) * r)
    lgc = jnp.log((p1 * q2) * z1 * r * jax.lax.rsqrt(t1 * t2))
    contrib = a1 * lga + xb * lgb + lgc
    out_ref[0, 0, :] = -(jnp.sum(contrib, axis=1) + c_ref[0, 0])


@functools.partial(jax.jit, static_argnames=("interpret",))
def _run(x, y, px_r, W, interpret=False):
    nb = BATCH // B_TILE
    y2 = y.reshape(nb, 1, B_TILE)
    pxr2 = px_r.reshape(1, N_INPUT)
    wt = W.T

    loss = pl.pallas_call(
        _loss_kernel,
        grid=(nb,),
        in_specs=[
            pl.BlockSpec(memory_space=pltpu.MemorySpace.HBM),
            pl.BlockSpec((1, 1, B_TILE), lambda i: (i, 0, 0)),
            pl.BlockSpec((1, N_INPUT), lambda i: (0, 0)),
            pl.BlockSpec((N_LABELS, N_INPUT), lambda i: (0, 0)),
        ],
        out_specs=pl.BlockSpec((1, 1, B_TILE), lambda i: (i, 0, 0)),
        out_shape=jax.ShapeDtypeStruct((nb, 1, B_TILE), jnp.float32),
        scratch_shapes=[
            pltpu.VMEM((2, B_TILE, N_INPUT), jnp.float32),
            pltpu.VMEM((3 * N_LABELS, N_INPUT), jnp.bfloat16),
            pltpu.VMEM((1, N_INPUT), jnp.float32),
            pltpu.SMEM((1, 1), jnp.float32),
            pltpu.SemaphoreType.DMA((2,)),
        ],
        interpret=interpret,
    )(x, y2, pxr2, wt)
    return loss.reshape(BATCH)


def kernel(x, y, ind_x, px_r, W):
    loss = _run(x, y, px_r, W)
    zero = jnp.asarray(0.0, dtype=jnp.float32)
    return (loss, zero, zero)
